# Initial kernel scaffold; baseline (speedup 1.0000x reference)
import jax
import jax.numpy as jnp
from jax.experimental import pallas as pl


def kernel(q, mem_k, mem_v, mem_m):
    raise NotImplementedError("write your pallas kernel here")


# speedup vs baseline: 1.3901x; 1.3901x over previous
import functools

import jax
import jax.numpy as jnp
from jax import lax
from jax.experimental import pallas as pl
from jax.experimental.pallas import tpu as pltpu
from jax.experimental.pallas import tpu_sc as plsc

_TOPK = 8
_LANES = 16
_NC, _NS = 2, 16
_NW = _NC * _NS


def _tc_stage(q, mem_k, mem_m, block_n):
    N, K, H = mem_k.shape
    grid = N // block_n

    def body(q_ref, mk_ref, mm_ref, gidx_ref, w_ref):
        bn = q_ref.shape[0]
        qv = q_ref[...]
        mk = mk_ref[...]
        mm = mm_ref[...]
        q2 = jnp.sum(qv * qv, axis=1, keepdims=True)
        qn = qv / jnp.maximum(jnp.sqrt(q2), 1e-12)
        qb = qn.astype(jnp.bfloat16)
        sq = mk * mk
        hi = sq.astype(jnp.bfloat16)
        lo = (sq - hi.astype(jnp.float32)).astype(jnp.bfloat16)
        ones = jnp.ones((H,), jnp.bfloat16)
        cdim = (((2,), (0,)), ((), ()))
        k2 = (lax.dot_general(hi, ones, cdim, preferred_element_type=jnp.float32)
              + lax.dot_general(lo, ones, cdim, preferred_element_type=jnp.float32))
        rk = 1.0 / jnp.maximum(jnp.sqrt(k2), 1e-12)
        kb = (mk * rk[:, :, None]).astype(jnp.bfloat16)
        d = lax.dot_general(kb, qb, (((2,), (1,)), ((0,), (0,))),
                            preferred_element_type=jnp.float32)
        sim = jnp.where(mm <= 0.0, jnp.float32(-1e9), d)
        kiota = lax.broadcasted_iota(jnp.int32, (bn, K), 1)
        vals, idxs = [], []
        s = sim
        for _ in range(_TOPK):
            m = jnp.max(s, axis=1, keepdims=True)
            sel = jnp.min(jnp.where(s == m, kiota, K), axis=1, keepdims=True)
            vals.append(m)
            idxs.append(sel)
            s = jnp.where(kiota == sel, jnp.float32(-jnp.inf), s)
        v8 = jnp.concatenate(vals, axis=1)
        i8 = jnp.concatenate(idxs, axis=1)
        mx = jnp.max(v8, axis=1, keepdims=True)
        e = jnp.exp(v8 - mx)
        attn = e / jnp.sum(e, axis=1, keepdims=True)
        pid = pl.program_id(0)
        rowi = lax.broadcasted_iota(jnp.int32, (bn, _TOPK), 0)
        gidx_ref[...] = (pid * bn + rowi) * K + i8
        liota = lax.broadcasted_iota(jnp.int32, (bn, H), 1) // _LANES
        w128 = jnp.zeros((bn, H), jnp.float32)
        for t in range(_TOPK):
            w128 = jnp.where(liota == t, jnp.broadcast_to(attn[:, t:t + 1], (bn, H)), w128)
        w_ref[...] = w128

    return pl.pallas_call(
        body,
        grid=(grid,),
        in_specs=[
            pl.BlockSpec((block_n, H), lambda i: (i, 0)),
            pl.BlockSpec((block_n, K, H), lambda i: (i, 0, 0)),
            pl.BlockSpec((block_n, K), lambda i: (i, 0)),
        ],
        out_specs=[
            pl.BlockSpec((block_n, _TOPK), lambda i: (i, 0)),
            pl.BlockSpec((block_n, H), lambda i: (i, 0)),
        ],
        out_shape=[
            jax.ShapeDtypeStruct((N, _TOPK), jnp.int32),
            jax.ShapeDtypeStruct((N, H), jnp.float32),
        ],
    )(q, mem_k, mem_m)


def _sc_stage(vflat, gidx2d, w128, N, H):
    n_per = N // _NW
    ch = 16
    n_chunks = n_per // ch
    mesh = plsc.VectorSubcoreMesh(core_axis_name="c", subcore_axis_name="s")

    @functools.partial(
        pl.kernel,
        out_type=jax.ShapeDtypeStruct((N, H), jnp.float32),
        mesh=mesh,
        scratch_types=[
            pltpu.VMEM((n_chunks, ch * _TOPK), jnp.int32),
            pltpu.VMEM((n_per, 128), jnp.float32),
            pltpu.VMEM((ch * _TOPK, H), jnp.float32),
            pltpu.VMEM((ch, H), jnp.float32),
            pltpu.SemaphoreType.DMA,
        ],
    )
    def run(vflat_hbm, gidx_hbm, w_hbm, out_hbm, idx_v, w_v, rows_v, out_v, sem):
        wid = lax.axis_index("s") * _NC + lax.axis_index("c")
        pltpu.sync_copy(gidx_hbm.at[pl.ds(wid * n_chunks, n_chunks)], idx_v)
        pltpu.sync_copy(w_hbm.at[pl.ds(wid * n_per, n_per)], w_v)
        for c in range(n_chunks):
            pltpu.async_copy(vflat_hbm.at[idx_v.at[c]], rows_v, sem).wait()

            def body(i, _, c=c):
                accs = [jnp.zeros((_LANES,), jnp.float32) for _ in range(H // _LANES)]
                for t in range(_TOPK):
                    wv = w_v[c * ch + i, pl.ds(t * _LANES, _LANES)]
                    for j in range(H // _LANES):
                        accs[j] = accs[j] + wv * rows_v[i * _TOPK + t, pl.ds(j * _LANES, _LANES)]
                for j in range(H // _LANES):
                    out_v[i, pl.ds(j * _LANES, _LANES)] = accs[j]
                return 0

            lax.fori_loop(0, ch, body, 0)
            pltpu.sync_copy(out_v, out_hbm.at[pl.ds(wid * n_per + c * ch, ch)])

    return run(vflat, gidx2d, w128)


def kernel(q, mem_k, mem_v, mem_m):
    N, K, H = mem_k.shape
    gidx, w128 = _tc_stage(q, mem_k, mem_m, block_n=32)
    vflat = mem_v.reshape(N * K, H)
    gidx2d = gidx.reshape(N * _TOPK // 128, 128)
    return _sc_stage(vflat, gidx2d, w128, N, H)


# speedup vs baseline: 2.1446x; 1.5427x over previous
import functools

import jax
import jax.numpy as jnp
from jax import lax
from jax.experimental import pallas as pl
from jax.experimental.pallas import tpu as pltpu
from jax.experimental.pallas import tpu_sc as plsc

_TOPK = 8
_LANES = 16
_NC, _NS = 2, 16
_NW = _NC * _NS


def _tc_stage(q, mem_k, mem_m, block_n):
    N, K, H = mem_k.shape
    grid = N // block_n

    def body(q_ref, mk_ref, mm_ref, gidx_ref, w_ref):
        bn = q_ref.shape[0]
        qv = q_ref[...]
        mk = mk_ref[...]
        mm = mm_ref[...]
        q2 = jnp.sum(qv * qv, axis=1, keepdims=True)
        qn = qv * lax.rsqrt(jnp.maximum(q2, 1e-24))
        qb = qn.astype(jnp.bfloat16)
        sq = mk * mk
        hi = sq.astype(jnp.bfloat16)
        lo = (sq - hi.astype(jnp.float32)).astype(jnp.bfloat16)
        ones = jnp.ones((H,), jnp.bfloat16)
        cdim = (((2,), (0,)), ((), ()))
        k2 = (lax.dot_general(hi, ones, cdim, preferred_element_type=jnp.float32)
              + lax.dot_general(lo, ones, cdim, preferred_element_type=jnp.float32))
        rk = lax.rsqrt(jnp.maximum(k2, 1e-24))
        kb = (mk * rk[:, :, None]).astype(jnp.bfloat16)
        d = lax.dot_general(kb, qb, (((2,), (1,)), ((0,), (0,))),
                            preferred_element_type=jnp.float32)
        sim = jnp.where(mm <= 0.0, jnp.float32(-1e9), d)
        kiota = lax.broadcasted_iota(jnp.int32, (bn, K), 1)
        vals, idxs = [], []
        s = sim
        for _ in range(_TOPK):
            m = jnp.max(s, axis=1, keepdims=True)
            sel = jnp.min(jnp.where(s == m, kiota, K), axis=1, keepdims=True)
            vals.append(m)
            idxs.append(sel)
            s = jnp.where(kiota == sel, jnp.float32(-jnp.inf), s)
        v8 = jnp.concatenate(vals, axis=1)
        i8 = jnp.concatenate(idxs, axis=1)
        mx = jnp.max(v8, axis=1, keepdims=True)
        e = jnp.exp(v8 - mx)
        attn = e / jnp.sum(e, axis=1, keepdims=True)
        pid = pl.program_id(0)
        rowi = lax.broadcasted_iota(jnp.int32, (bn, _TOPK), 0)
        gidx_ref[...] = (pid * bn + rowi) * K + i8
        liota = lax.broadcasted_iota(jnp.int32, (bn, H), 1) // _LANES
        w128 = jnp.zeros((bn, H), jnp.float32)
        for t in range(_TOPK):
            w128 = jnp.where(liota == t, jnp.broadcast_to(attn[:, t:t + 1], (bn, H)), w128)
        w_ref[...] = w128

    return pl.pallas_call(
        body,
        grid=(grid,),
        in_specs=[
            pl.BlockSpec((block_n, H), lambda i: (i, 0)),
            pl.BlockSpec((block_n, K, H), lambda i: (i, 0, 0)),
            pl.BlockSpec((block_n, K), lambda i: (i, 0)),
        ],
        out_specs=[
            pl.BlockSpec((block_n, _TOPK), lambda i: (i, 0)),
            pl.BlockSpec((block_n, H), lambda i: (i, 0)),
        ],
        out_shape=[
            jax.ShapeDtypeStruct((N, _TOPK), jnp.int32),
            jax.ShapeDtypeStruct((N, H), jnp.float32),
        ],
    )(q, mem_k, mem_m)


def _sc_stage(vflat, gidx2d, w128, N, H):
    n_per = N // _NW
    ch = 16
    n_chunks = n_per // ch
    mesh = plsc.VectorSubcoreMesh(core_axis_name="c", subcore_axis_name="s")

    @functools.partial(
        pl.kernel,
        out_type=jax.ShapeDtypeStruct((N, H), jnp.float32),
        mesh=mesh,
        scratch_types=[
            pltpu.VMEM((n_chunks, ch * _TOPK), jnp.int32),
            pltpu.VMEM((n_per, 128), jnp.float32),
            pltpu.VMEM((ch * _TOPK, H), jnp.float32),
            pltpu.VMEM((ch, H), jnp.float32),
            pltpu.SemaphoreType.DMA,
        ],
    )
    def run(vflat_hbm, gidx_hbm, w_hbm, out_hbm, idx_v, w_v, rows_v, out_v, sem):
        wid = lax.axis_index("s") * _NC + lax.axis_index("c")
        pltpu.sync_copy(gidx_hbm.at[pl.ds(wid * n_chunks, n_chunks)], idx_v)
        pltpu.sync_copy(w_hbm.at[pl.ds(wid * n_per, n_per)], w_v)
        for c in range(n_chunks):
            pltpu.async_copy(vflat_hbm.at[idx_v.at[c]], rows_v, sem).wait()

            def body(i, _, c=c):
                accs = [jnp.zeros((_LANES,), jnp.float32) for _ in range(H // _LANES)]
                for t in range(_TOPK):
                    wv = w_v[c * ch + i, pl.ds(t * _LANES, _LANES)]
                    for j in range(H // _LANES):
                        accs[j] = accs[j] + wv * rows_v[i * _TOPK + t, pl.ds(j * _LANES, _LANES)]
                for j in range(H // _LANES):
                    out_v[i, pl.ds(j * _LANES, _LANES)] = accs[j]
                return 0

            lax.fori_loop(0, ch, body, 0)
            pltpu.sync_copy(out_v, out_hbm.at[pl.ds(wid * n_per + c * ch, ch)])

    return run(vflat, gidx2d, w128)


def kernel(q, mem_k, mem_v, mem_m):
    N, K, H = mem_k.shape
    gidx, w128 = _tc_stage(q, mem_k, mem_m, block_n=128)
    vflat = mem_v.reshape(N * K, H)
    gidx2d = gidx.reshape(N * _TOPK // 128, 128)
    return _sc_stage(vflat, gidx2d, w128, N, H)


# speedup vs baseline: 2.9718x; 1.3857x over previous
import functools

import jax
import jax.numpy as jnp
from jax import lax
from jax.experimental import pallas as pl
from jax.experimental.pallas import tpu as pltpu
from jax.experimental.pallas import tpu_sc as plsc

_TOPK = 8
_LANES = 16
_NC, _NS = 2, 16
_NW = _NC * _NS


def _tc_stage(q, mem_k, mem_m, block_n):
    N, K, H = mem_k.shape
    grid = N // block_n

    def body(q_ref, mk_ref, mm_ref, gidx_ref, w_ref):
        bn = q_ref.shape[0]
        qv = q_ref[...]
        mk = mk_ref[...]
        mm = mm_ref[...]
        q2 = jnp.sum(qv * qv, axis=1, keepdims=True)
        qn = qv * lax.rsqrt(jnp.maximum(q2, 1e-24))
        qb = qn.astype(jnp.bfloat16)
        ones = jnp.ones((H,), jnp.float32)
        cdim = (((2,), (0,)), ((), ()))
        sq = mk * mk
        k2 = lax.dot_general(sq, ones, cdim, precision=lax.Precision.HIGHEST,
                             preferred_element_type=jnp.float32)
        rk = lax.rsqrt(jnp.maximum(k2, 1e-24))
        kb = (mk * rk[:, :, None]).astype(jnp.bfloat16)
        d = lax.dot_general(kb, qb, (((2,), (1,)), ((0,), (0,))),
                            preferred_element_type=jnp.float32)
        sim = jnp.where(mm <= 0.0, jnp.float32(-1e9), d)
        kiota = lax.broadcasted_iota(jnp.int32, (bn, K), 1)
        vals, idxs = [], []
        s = sim
        for _ in range(_TOPK):
            m = jnp.max(s, axis=1, keepdims=True)
            sel = jnp.min(jnp.where(s == m, kiota, K), axis=1, keepdims=True)
            vals.append(m)
            idxs.append(sel)
            s = jnp.where(kiota == sel, jnp.float32(-jnp.inf), s)
        v8 = jnp.concatenate(vals, axis=1)
        i8 = jnp.concatenate(idxs, axis=1)
        mx = jnp.max(v8, axis=1, keepdims=True)
        e = jnp.exp(v8 - mx)
        attn = e / jnp.sum(e, axis=1, keepdims=True)
        pid = pl.program_id(0)
        rowi = lax.broadcasted_iota(jnp.int32, (bn, _TOPK), 0)
        gidx_ref[...] = (pid * bn + rowi) * K + i8
        liota = lax.broadcasted_iota(jnp.int32, (bn, H), 1) // _LANES
        w128 = jnp.zeros((bn, H), jnp.float32)
        for t in range(_TOPK):
            w128 = jnp.where(liota == t, jnp.broadcast_to(attn[:, t:t + 1], (bn, H)), w128)
        w_ref[...] = w128

    return pl.pallas_call(
        body,
        grid=(grid,),
        in_specs=[
            pl.BlockSpec((block_n, H), lambda i: (i, 0)),
            pl.BlockSpec((block_n, K, H), lambda i: (i, 0, 0)),
            pl.BlockSpec((block_n, K), lambda i: (i, 0)),
        ],
        out_specs=[
            pl.BlockSpec((block_n, _TOPK), lambda i: (i, 0)),
            pl.BlockSpec((block_n, H), lambda i: (i, 0)),
        ],
        out_shape=[
            jax.ShapeDtypeStruct((N, _TOPK), jnp.int32),
            jax.ShapeDtypeStruct((N, H), jnp.float32),
        ],
    )(q, mem_k, mem_m)


def _sc_stage(vflat, gidx2d, w128, N, H):
    n_per = N // _NW
    ch = 16
    n_chunks = n_per // ch
    mesh = plsc.VectorSubcoreMesh(core_axis_name="c", subcore_axis_name="s")

    @functools.partial(
        pl.kernel,
        out_type=jax.ShapeDtypeStruct((N, H), jnp.float32),
        mesh=mesh,
        scratch_types=[
            pltpu.VMEM((n_chunks, ch * _TOPK), jnp.int32),
            pltpu.VMEM((n_per, 128), jnp.float32),
            pltpu.VMEM((ch * _TOPK, H), jnp.float32),
            pltpu.VMEM((ch, H), jnp.float32),
            pltpu.SemaphoreType.DMA,
        ],
    )
    def run(vflat_hbm, gidx_hbm, w_hbm, out_hbm, idx_v, w_v, rows_v, out_v, sem):
        wid = lax.axis_index("s") * _NC + lax.axis_index("c")
        pltpu.sync_copy(gidx_hbm.at[pl.ds(wid * n_chunks, n_chunks)], idx_v)
        pltpu.sync_copy(w_hbm.at[pl.ds(wid * n_per, n_per)], w_v)
        for c in range(n_chunks):
            pltpu.async_copy(vflat_hbm.at[idx_v.at[c]], rows_v, sem).wait()

            def body(i, _, c=c):
                accs = [jnp.zeros((_LANES,), jnp.float32) for _ in range(H // _LANES)]
                for t in range(_TOPK):
                    wv = w_v[c * ch + i, pl.ds(t * _LANES, _LANES)]
                    for j in range(H // _LANES):
                        accs[j] = accs[j] + wv * rows_v[i * _TOPK + t, pl.ds(j * _LANES, _LANES)]
                for j in range(H // _LANES):
                    out_v[i, pl.ds(j * _LANES, _LANES)] = accs[j]
                return 0

            lax.fori_loop(0, ch, body, 0)
            pltpu.sync_copy(out_v, out_hbm.at[pl.ds(wid * n_per + c * ch, ch)])

    return run(vflat, gidx2d, w128)


def kernel(q, mem_k, mem_v, mem_m):
    N, K, H = mem_k.shape
    gidx, w128 = _tc_stage(q, mem_k, mem_m, block_n=128)
    vflat = mem_v.reshape(N * K, H)
    gidx2d = gidx.reshape(N * _TOPK // 128, 128)
    return _sc_stage(vflat, gidx2d, w128, N, H)


# speedup vs baseline: 3.4206x; 1.1510x over previous
import functools

import jax
import jax.numpy as jnp
from jax import lax
from jax.experimental import pallas as pl
from jax.experimental.pallas import tpu as pltpu
from jax.experimental.pallas import tpu_sc as plsc

_TOPK = 8
_LANES = 16
_NC, _NS = 2, 16
_NW = _NC * _NS


def _tc_stage(q, mem_k, mem_m, block_n):
    N, K, H = mem_k.shape
    grid = N // block_n

    def body(q_ref, mk_ref, mm_ref, gidx_ref, w_ref):
        bn = q_ref.shape[0]
        qv = q_ref[...]
        mk = mk_ref[...]
        mm = mm_ref[...]
        q2 = jnp.sum(qv * qv, axis=1, keepdims=True)
        qn = qv * lax.rsqrt(jnp.maximum(q2, 1e-24))
        qb = qn.astype(jnp.bfloat16)
        ones = jnp.ones((H,), jnp.float32)
        cdim = (((2,), (0,)), ((), ()))
        sq = mk * mk
        k2 = lax.dot_general(sq, ones, cdim, precision=lax.Precision.HIGHEST,
                             preferred_element_type=jnp.float32)
        rk = lax.rsqrt(jnp.maximum(k2, 1e-24))
        kb = (mk * rk[:, :, None]).astype(jnp.bfloat16)
        d = lax.dot_general(kb, qb, (((2,), (1,)), ((0,), (0,))),
                            preferred_element_type=jnp.float32)
        sim = jnp.where(mm <= 0.0, jnp.float32(-1e9), d)
        sT = sim.T
        kiota = lax.broadcasted_iota(jnp.int32, (K, bn), 0)
        vals, idxs = [], []
        for _ in range(_TOPK):
            m = jnp.max(sT, axis=0, keepdims=True)
            sel = jnp.min(jnp.where(sT == m, kiota, K), axis=0, keepdims=True)
            vals.append(m)
            idxs.append(sel)
            sT = jnp.where(kiota == sel, jnp.float32(-jnp.inf), sT)
        v8T = jnp.concatenate(vals, axis=0)
        i8 = jnp.concatenate(idxs, axis=0).T
        mxT = jnp.max(v8T, axis=0, keepdims=True)
        eT = jnp.exp(v8T - mxT)
        attn = (eT / jnp.sum(eT, axis=0, keepdims=True)).T
        pid = pl.program_id(0)
        rowi = lax.broadcasted_iota(jnp.int32, (bn, _TOPK), 0)
        gidx_ref[...] = (pid * bn + rowi) * K + i8
        liota = lax.broadcasted_iota(jnp.int32, (bn, H), 1) // _LANES
        w128 = jnp.zeros((bn, H), jnp.float32)
        for t in range(_TOPK):
            w128 = jnp.where(liota == t, jnp.broadcast_to(attn[:, t:t + 1], (bn, H)), w128)
        w_ref[...] = w128

    return pl.pallas_call(
        body,
        grid=(grid,),
        in_specs=[
            pl.BlockSpec((block_n, H), lambda i: (i, 0)),
            pl.BlockSpec((block_n, K, H), lambda i: (i, 0, 0)),
            pl.BlockSpec((block_n, K), lambda i: (i, 0)),
        ],
        out_specs=[
            pl.BlockSpec((block_n, _TOPK), lambda i: (i, 0)),
            pl.BlockSpec((block_n, H), lambda i: (i, 0)),
        ],
        out_shape=[
            jax.ShapeDtypeStruct((N, _TOPK), jnp.int32),
            jax.ShapeDtypeStruct((N, H), jnp.float32),
        ],
    )(q, mem_k, mem_m)


def _sc_stage(vflat, gidx2d, w128, N, H):
    n_per = N // _NW
    ch = 16
    n_chunks = n_per // ch
    mesh = plsc.VectorSubcoreMesh(core_axis_name="c", subcore_axis_name="s")

    @functools.partial(
        pl.kernel,
        out_type=jax.ShapeDtypeStruct((N, H), jnp.float32),
        mesh=mesh,
        scratch_types=[
            pltpu.VMEM((n_chunks, ch * _TOPK), jnp.int32),
            pltpu.VMEM((n_per, 128), jnp.float32),
            pltpu.VMEM((ch * _TOPK, H), jnp.float32),
            pltpu.VMEM((ch, H), jnp.float32),
            pltpu.SemaphoreType.DMA,
        ],
    )
    def run(vflat_hbm, gidx_hbm, w_hbm, out_hbm, idx_v, w_v, rows_v, out_v, sem):
        wid = lax.axis_index("s") * _NC + lax.axis_index("c")
        pltpu.sync_copy(gidx_hbm.at[pl.ds(wid * n_chunks, n_chunks)], idx_v)
        pltpu.sync_copy(w_hbm.at[pl.ds(wid * n_per, n_per)], w_v)
        for c in range(n_chunks):
            pltpu.async_copy(vflat_hbm.at[idx_v.at[c]], rows_v, sem).wait()

            def body(i, _, c=c):
                accs = [jnp.zeros((_LANES,), jnp.float32) for _ in range(H // _LANES)]
                for t in range(_TOPK):
                    wv = w_v[c * ch + i, pl.ds(t * _LANES, _LANES)]
                    for j in range(H // _LANES):
                        accs[j] = accs[j] + wv * rows_v[i * _TOPK + t, pl.ds(j * _LANES, _LANES)]
                for j in range(H // _LANES):
                    out_v[i, pl.ds(j * _LANES, _LANES)] = accs[j]
                return 0

            lax.fori_loop(0, ch, body, 0)
            pltpu.sync_copy(out_v, out_hbm.at[pl.ds(wid * n_per + c * ch, ch)])

    return run(vflat, gidx2d, w128)


def kernel(q, mem_k, mem_v, mem_m):
    N, K, H = mem_k.shape
    gidx, w128 = _tc_stage(q, mem_k, mem_m, block_n=128)
    vflat = mem_v.reshape(N * K, H)
    gidx2d = gidx.reshape(N * _TOPK // 128, 128)
    return _sc_stage(vflat, gidx2d, w128, N, H)
